# Initial kernel scaffold; baseline (speedup 1.0000x reference)
import jax
import jax.numpy as jnp
from jax.experimental import pallas as pl


def kernel(pred_rot_matrix, pred_trans, target_rot_matrix, target_trans, fg_mask, class_ids, model_points):
    raise NotImplementedError("write your pallas kernel here")


# speedup vs baseline: 7.6329x; 7.6329x over previous
import jax
import jax.numpy as jnp
from jax import lax
from jax.experimental import pallas as pl
from jax.experimental.pallas import tpu as pltpu

_PP = 1024
_BIG = 1e30


def _adds_body(cls_ref, valid_ref, rp_ref, tp_ref, rg_ref, tg_ref,
               model_ref, validf_ref, out_ref, A, G, GTMIN, PMIN, ACC):
    M = cls_ref.shape[0]
    P = 1000
    lane_q = lax.broadcasted_iota(jnp.int32, (1, _PP), 1)
    padmask = lane_q >= P
    ones_row = jnp.ones((1, _PP), jnp.float32)
    zeros3 = jnp.zeros((3, _PP), jnp.float32)

    def build(m, carry):
        c = cls_ref[m]
        pts = model_ref[c]
        x = pts[0:1, :]
        y = pts[1:2, :]
        z = pts[2:3, :]
        px = rp_ref[m, 0] * x + rp_ref[m, 1] * y + rp_ref[m, 2] * z + tp_ref[m, 0]
        py = rp_ref[m, 3] * x + rp_ref[m, 4] * y + rp_ref[m, 5] * z + tp_ref[m, 1]
        pz = rp_ref[m, 6] * x + rp_ref[m, 7] * y + rp_ref[m, 8] * z + tp_ref[m, 2]
        pf2 = jnp.where(padmask, _BIG, px * px + py * py + pz * pz)
        A[m] = jnp.concatenate(
            [-2.0 * px, -2.0 * py, -2.0 * pz, pf2, ones_row, zeros3], axis=0)
        gx = rg_ref[m, 0] * x + rg_ref[m, 1] * y + rg_ref[m, 2] * z + tg_ref[m, 0]
        gy = rg_ref[m, 3] * x + rg_ref[m, 4] * y + rg_ref[m, 5] * z + tg_ref[m, 1]
        gz = rg_ref[m, 6] * x + rg_ref[m, 7] * y + rg_ref[m, 8] * z + tg_ref[m, 2]
        gf2 = jnp.where(padmask, _BIG, gx * gx + gy * gy + gz * gz)
        G[m] = jnp.concatenate(
            [gx, gy, gz, ones_row, gf2, zeros3], axis=0)
        GTMIN[pl.ds(m, 1), :] = jnp.full((1, _PP), _BIG, jnp.float32)
        return carry

    lax.fori_loop(0, M, build, 0)
    ACC[0] = jnp.float32(0.0)

    def pair_m(m, carry):
        @pl.when(valid_ref[m] != 0)
        def _():
            a = A[m]
            cm = cls_ref[m]
            PMIN[:] = jnp.full((_PP, 1), _BIG, jnp.float32)

            def pair_n(n, c2):
                match = (valid_ref[n] != 0) & (cls_ref[n] == cm)

                @pl.when(match)
                def _():
                    g = G[n]
                    d2 = lax.dot_general(
                        a, g, (((0,), (0,)), ((), ())),
                        preferred_element_type=jnp.float32)
                    PMIN[:] = jnp.minimum(PMIN[:], jnp.min(d2, axis=1, keepdims=True))
                    GTMIN[pl.ds(n, 1), :] = jnp.minimum(
                        GTMIN[pl.ds(n, 1), :], jnp.min(d2, axis=0, keepdims=True))
                return c2

            lax.fori_loop(0, M, pair_n, 0)
            spm = jnp.sqrt(jnp.maximum(PMIN[:], 0.0))
            row_p = lax.broadcasted_iota(jnp.int32, (_PP, 1), 0)
            spm = jnp.where(row_p >= P, 0.0, spm)
            ACC[0] = ACC[0] + jnp.sum(spm)
        return carry

    lax.fori_loop(0, M, pair_m, 0)

    sg = jnp.sqrt(jnp.maximum(GTMIN[:], 0.0))
    qmask = lax.broadcasted_iota(jnp.int32, (M, _PP), 1) < P
    vmask = validf_ref[:] > 0.0
    sg = jnp.where(qmask & vmask, sg, 0.0)
    total = ACC[0] + jnp.sum(sg)
    cnt = jnp.sum(validf_ref[:])
    res = jnp.where(cnt == 0.0, jnp.float32(0.0),
                    total / (2.0 * P * jnp.maximum(cnt, 1.0)))
    out_ref[...] = jnp.full((1, 1), res, jnp.float32)


def kernel(pred_rot_matrix, pred_trans, target_rot_matrix, target_trans,
           fg_mask, class_ids, model_points):
    B, N = fg_mask.shape
    M = B * N
    C, P, _ = model_points.shape
    rp = pred_rot_matrix.reshape(M, 9).astype(jnp.float32)
    tp = pred_trans.reshape(M, 3).astype(jnp.float32)
    rg = target_rot_matrix.reshape(M, 9).astype(jnp.float32)
    tg = target_trans.reshape(M, 3).astype(jnp.float32)
    cls = class_ids.reshape(M).astype(jnp.int32)
    valid = fg_mask.reshape(M).astype(jnp.int32)
    validf = fg_mask.reshape(M, 1).astype(jnp.float32)
    mp = jnp.transpose(model_points.astype(jnp.float32), (0, 2, 1))
    mp = jnp.pad(mp, ((0, 0), (0, 0), (0, _PP - P)))

    smem = pl.BlockSpec(memory_space=pltpu.SMEM)
    vmem = pl.BlockSpec(memory_space=pltpu.VMEM)
    out = pl.pallas_call(
        _adds_body,
        in_specs=[smem, smem, smem, smem, smem, smem, vmem, vmem],
        out_specs=vmem,
        out_shape=jax.ShapeDtypeStruct((1, 1), jnp.float32),
        scratch_shapes=[
            pltpu.VMEM((M, 8, _PP), jnp.float32),
            pltpu.VMEM((M, 8, _PP), jnp.float32),
            pltpu.VMEM((M, _PP), jnp.float32),
            pltpu.VMEM((_PP, 1), jnp.float32),
            pltpu.SMEM((1,), jnp.float32),
        ],
    )(cls, valid, rp, tp, rg, tg, mp, validf)
    return jax.lax.stop_gradient(out[0, 0])
